# baseline (speedup 1.0000x reference)
import functools
import math

import jax
import jax.numpy as jnp
from jax import lax
from jax.experimental import pallas as pl
from jax.experimental.pallas import tpu as pltpu
from jax.experimental.pallas import tpu_sc as plsc

D_MODEL = 64
VOCAB = 1_000_000
SCALE = math.sqrt(D_MODEL)

WIN = 4096
HW = WIN // 2
NG = -(-VOCAB // WIN)
VP = NG * WIN

NUM_CORES = 2
NUM_SUBCORES = 16
LANES = 16
NW = NUM_CORES * NUM_SUBCORES

B1 = 4096
J = 200
STRIPE = B1 // NW

def _t2_body(a_ref, o_ref):
    ia = lax.broadcasted_iota(jnp.int32, (D_MODEL, D_MODEL), 0)
    ib = lax.broadcasted_iota(jnp.int32, (D_MODEL, D_MODEL), 1)
    eye8 = jnp.where(ia == ib, SCALE, 0.0).astype(jnp.float32)
    dn = (((0,), (0,)), ((), ()))
    o_ref[:, 0:D_MODEL] = lax.dot_general(
        a_ref[:, 0:HW], eye8, dn,
        precision=lax.Precision.HIGHEST,
        preferred_element_type=jnp.float32)
    o_ref[:, D_MODEL:2 * D_MODEL] = lax.dot_general(
        a_ref[:, HW:WIN], eye8, dn,
        precision=lax.Precision.HIGHEST,
        preferred_element_type=jnp.float32)


def _build_t2(tt):
    return pl.pallas_call(
        _t2_body,
        grid=(NG,),
        in_specs=[pl.BlockSpec((D_MODEL, WIN), lambda g: (0, g))],
        out_specs=pl.BlockSpec((HW, 2 * D_MODEL), lambda g: (g, 0)),
        out_shape=jax.ShapeDtypeStruct((VP // 2, 2 * D_MODEL), jnp.float32),
    )(tt)


C = 256
CPP = B1 // C
CH = (J * CPP) // NW
NPAIR = CH // 2


def _sc_body(u_hbm, t_hbm, out_hbm,
             u0, u1, rows0, rows1, to0, to1,
             si0, si1, sg0, sg1, ss0, ss1):
    wid = lax.axis_index("s") * NUM_CORES + lax.axis_index("c")
    h0 = wid * CH

    def idx_copy(g, u_v, sem):
        off = pl.multiple_of((h0 + g) * C, C)
        return pltpu.make_async_copy(u_hbm.at[pl.ds(off, C)], u_v, sem)

    def gather(u_v, rows_v, sem):
        return pltpu.make_async_copy(t_hbm.at[u_v], rows_v, sem)

    def store(g, to_v, sem):
        h = h0 + g
        j = h >> 4
        tc0 = (h & (CPP - 1)) * (C // 128)
        return pltpu.make_async_copy(
            to_v, out_hbm.at[j, :, pl.ds(tc0, C // 128), :, :], sem)

    def transpose(rows_v, to_v):
        iota = lax.iota(jnp.int32, LANES)

        @pl.loop(0, D_MODEL, unroll=4)
        def _(s):
            dv = jnp.bitwise_and(iota + s, D_MODEL - 1)
            band_v = dv >> 3
            row_v = jnp.bitwise_and(dv, 7)
            for i in range(C // LANES):
                tv = iota + (LANES * i)
                tc_v = jnp.full((LANES,), (LANES * i) >> 7, jnp.int32)
                col_v = iota + ((LANES * i) & 127)
                vals = plsc.load_gather(rows_v, [tv, dv])
                plsc.store_scatter(to_v, [band_v, tc_v, row_v, col_v], vals)

    idx_copy(0, u0, si0).start()
    idx_copy(0, u0, si0).wait()
    gather(u0, rows0, sg0).start()
    idx_copy(1, u1, si1).start()

    @pl.loop(0, NPAIR)
    def _(p):
        a = p * 2

        idx_copy(a + 1, u1, si1).wait()
        gather(u1, rows1, sg1).start()
        gather(u0, rows0, sg0).wait()

        @pl.when(p < NPAIR - 1)
        def _():
            idx_copy(a + 2, u0, si0).start()

        @pl.when(p > 0)
        def _():
            store(a, to0, ss0).wait()

        transpose(rows0, to0)
        store(a, to0, ss0).start()

        @pl.when(p < NPAIR - 1)
        def _():
            idx_copy(a + 2, u0, si0).wait()
            gather(u0, rows0, sg0).start()

        gather(u1, rows1, sg1).wait()

        @pl.when(p < NPAIR - 1)
        def _():
            idx_copy(a + 3, u1, si1).start()

        @pl.when(p > 0)
        def _():
            store(a + 1, to1, ss1).wait()

        transpose(rows1, to1)
        store(a + 1, to1, ss1).start()

    store(CH - 2, to0, ss0).wait()
    store(CH - 1, to1, ss1).wait()


def _sc_gather(u, t_lin):
    mesh = plsc.VectorSubcoreMesh(
        core_axis_name="c", subcore_axis_name="s",
        num_cores=NUM_CORES, num_subcores=NUM_SUBCORES)
    f = pl.kernel(
        _sc_body,
        out_type=jax.ShapeDtypeStruct(
            (J, D_MODEL // 8, B1 // 128, 8, 128), jnp.float32),
        mesh=mesh,
        scratch_types=[
            pltpu.VMEM((C,), jnp.int32),
            pltpu.VMEM((C,), jnp.int32),
            pltpu.VMEM((C, D_MODEL), jnp.float32),
            pltpu.VMEM((C, D_MODEL), jnp.float32),
            pltpu.VMEM((D_MODEL // 8, C // 128, 8, 128), jnp.float32),
            pltpu.VMEM((D_MODEL // 8, C // 128, 8, 128), jnp.float32),
            pltpu.SemaphoreType.DMA,
            pltpu.SemaphoreType.DMA,
            pltpu.SemaphoreType.DMA,
            pltpu.SemaphoreType.DMA,
            pltpu.SemaphoreType.DMA,
            pltpu.SemaphoreType.DMA,
        ],
        compiler_params=pltpu.CompilerParams(
            use_tc_tiling_on_sc=False, needs_layout_passes=False,
            disable_bounds_checks=True),
    )
    return f(u, t_lin)


def kernel(x, table):
    xt = x.T.reshape(-1).astype(jnp.int32)
    q = jnp.bitwise_and(xt, WIN - 1)
    u = (xt - q) + 2 * jnp.bitwise_and(q, HW - 1) + (q >> int(math.log2(HW)))
    t2 = _build_t2(table.T)
    t_lin = t2.reshape(VP, D_MODEL)
    out5d = _sc_gather(u, t_lin)
    return jnp.transpose(out5d, (2, 4, 0, 1, 3)).reshape(B1, J, D_MODEL)


# speedup vs baseline: 1.2546x; 1.2546x over previous
import functools
import math

import jax
import jax.numpy as jnp
from jax import lax
from jax.experimental import pallas as pl
from jax.experimental.pallas import tpu as pltpu
from jax.experimental.pallas import tpu_sc as plsc

D_MODEL = 64
VOCAB = 1_000_000
SCALE = math.sqrt(D_MODEL)

WIN = 4096
HW = WIN // 2
NG = -(-VOCAB // WIN)
VP = NG * WIN

NUM_CORES = 2
NUM_SUBCORES = 16
LANES = 16
NW = NUM_CORES * NUM_SUBCORES

B1 = 4096
J = 200
STRIPE = B1 // NW

def _t2_body(a_ref, o_ref):
    o_ref[:, 0:D_MODEL] = jnp.swapaxes(a_ref[:, 0:HW], 0, 1) * SCALE
    o_ref[:, D_MODEL:2 * D_MODEL] = jnp.swapaxes(a_ref[:, HW:WIN], 0, 1) * SCALE


def _build_t2(tt):
    return pl.pallas_call(
        _t2_body,
        grid=(NG,),
        in_specs=[pl.BlockSpec((D_MODEL, WIN), lambda g: (0, g))],
        out_specs=pl.BlockSpec((HW, 2 * D_MODEL), lambda g: (g, 0)),
        out_shape=jax.ShapeDtypeStruct((VP // 2, 2 * D_MODEL), jnp.float32),
    )(tt)


C = 256
CPP = B1 // C
CH = (J * CPP) // NW
NPAIR = CH // 2


def _sc_body(u_hbm, t_hbm, out_hbm,
             u0, u1, u2, u3, rows0, rows1, rows2, rows3, to0, to1,
             si0, si1, si2, si3, sg0, sg1, sg2, sg3, ss0, ss1):
    wid = lax.axis_index("s") * NUM_CORES + lax.axis_index("c")
    h0 = wid * CH
    ub = (u0, u1, u2, u3)
    rb = (rows0, rows1, rows2, rows3)
    tb = (to0, to1)
    sib = (si0, si1, si2, si3)
    sgb = (sg0, sg1, sg2, sg3)
    ssb = (ss0, ss1)

    def idx_copy(g, u_v, sem):
        off = pl.multiple_of((h0 + g) * C, C)
        return pltpu.make_async_copy(u_hbm.at[pl.ds(off, C)], u_v, sem)

    def gather(u_v, rows_v, sem):
        return pltpu.make_async_copy(t_hbm.at[u_v], rows_v, sem)

    def store(g, to_v, sem):
        h = h0 + g
        j = h >> 4
        tc0 = (h & (CPP - 1)) * (C // 128)
        return pltpu.make_async_copy(
            to_v, out_hbm.at[j, :, pl.ds(tc0, C // 128), :, :], sem)

    def transpose(rows_v, to_v):
        iota = lax.iota(jnp.int32, LANES)

        @pl.loop(0, D_MODEL, unroll=4)
        def _(s):
            dv = jnp.bitwise_and(iota + s, D_MODEL - 1)
            band_v = dv >> 3
            row_v = jnp.bitwise_and(dv, 7)
            for i in range(C // LANES):
                tv = iota + (LANES * i)
                tc_v = jnp.full((LANES,), (LANES * i) >> 7, jnp.int32)
                col_v = iota + ((LANES * i) & 127)
                vals = plsc.load_gather(rows_v, [tv, dv])
                plsc.store_scatter(to_v, [band_v, tc_v, row_v, col_v], vals)

    for k in range(4):
        idx_copy(k, ub[k], sib[k]).start()
    for k in range(3):
        idx_copy(k, ub[k], sib[k]).wait()
        gather(ub[k], rb[k], sgb[k]).start()

    NQ = CH // 4

    @pl.loop(0, NQ)
    def _(q):
        a = q * 4
        for k in range(4):
            g = a + k
            gather(ub[k], rb[k], sgb[k]).wait()

            @pl.when(q < NQ - 1)
            def _():
                idx_copy(g + 4, ub[k], sib[k]).start()

            if k < 2:
                @pl.when(q > 0)
                def _():
                    store(g - 2, tb[k & 1], ssb[k & 1]).wait()
            else:
                store(g - 2, tb[k & 1], ssb[k & 1]).wait()

            transpose(rb[k], tb[k & 1])
            store(g, tb[k & 1], ssb[k & 1]).start()

            nk = (k + 3) & 3

            @pl.when(g + 3 < CH)
            def _():
                idx_copy(g + 3, ub[nk], sib[nk]).wait()
                gather(ub[nk], rb[nk], sgb[nk]).start()

    store(CH - 2, tb[0], ssb[0]).wait()
    store(CH - 1, tb[1], ssb[1]).wait()


def _sc_gather(u, t_lin):
    mesh = plsc.VectorSubcoreMesh(
        core_axis_name="c", subcore_axis_name="s",
        num_cores=NUM_CORES, num_subcores=NUM_SUBCORES)
    f = pl.kernel(
        _sc_body,
        out_type=jax.ShapeDtypeStruct(
            (J, D_MODEL // 8, B1 // 128, 8, 128), jnp.float32),
        mesh=mesh,
        scratch_types=(
            [pltpu.VMEM((C,), jnp.int32)] * 4
            + [pltpu.VMEM((C, D_MODEL), jnp.float32)] * 4
            + [pltpu.VMEM((D_MODEL // 8, C // 128, 8, 128), jnp.float32)] * 2
            + [pltpu.SemaphoreType.DMA] * 10
        ),
        compiler_params=pltpu.CompilerParams(
            use_tc_tiling_on_sc=False, needs_layout_passes=False,
            disable_bounds_checks=True),
    )
    return f(u, t_lin)


def kernel(x, table):
    xt = x.T.reshape(-1).astype(jnp.int32)
    q = jnp.bitwise_and(xt, WIN - 1)
    u = (xt - q) + 2 * jnp.bitwise_and(q, HW - 1) + (q >> int(math.log2(HW)))
    t2 = _build_t2(table.T)
    t_lin = t2.reshape(VP, D_MODEL)
    out5d = _sc_gather(u, t_lin)
    return jnp.transpose(out5d, (2, 4, 0, 1, 3)).reshape(B1, J, D_MODEL)


# speedup vs baseline: 1.9500x; 1.5543x over previous
import functools
import math

import jax
import jax.numpy as jnp
from jax import lax
from jax.experimental import pallas as pl
from jax.experimental.pallas import tpu as pltpu
from jax.experimental.pallas import tpu_sc as plsc

D_MODEL = 64
VOCAB = 1_000_000
SCALE = math.sqrt(D_MODEL)

WIN = 4096
HW = WIN // 2
NG = -(-VOCAB // WIN)
VP = NG * WIN

NUM_CORES = 2
NUM_SUBCORES = 16
LANES = 16
NW = NUM_CORES * NUM_SUBCORES

B1 = 4096
J = 200
STRIPE = B1 // NW

def _t2_body(a_ref, o_ref):
    o_ref[:, 0:D_MODEL] = jnp.swapaxes(a_ref[:, 0:HW], 0, 1) * SCALE
    o_ref[:, D_MODEL:2 * D_MODEL] = jnp.swapaxes(a_ref[:, HW:WIN], 0, 1) * SCALE


def _build_t2(tt):
    return pl.pallas_call(
        _t2_body,
        grid=(NG,),
        in_specs=[pl.BlockSpec((D_MODEL, WIN), lambda g: (0, g))],
        out_specs=pl.BlockSpec((HW, 2 * D_MODEL), lambda g: (g, 0)),
        out_shape=jax.ShapeDtypeStruct((VP // 2, 2 * D_MODEL), jnp.float32),
    )(tt)


C = 256
CPP = B1 // C
CH = (J * CPP) // NW
NPAIR = CH // 2


def _sc_body(u_hbm, t_hbm, out_hbm,
             u0, u1, u2, u3, rows0, rows1, rows2, rows3, to0, to1,
             si0, si1, si2, si3, sg0, sg1, sg2, sg3, ss0, ss1):
    wid = lax.axis_index("s") * NUM_CORES + lax.axis_index("c")
    h0 = wid * CH
    ub = (u0, u1, u2, u3)
    rb = (rows0, rows1, rows2, rows3)
    tb = (to0, to1)
    sib = (si0, si1, si2, si3)
    sgb = (sg0, sg1, sg2, sg3)
    ssb = (ss0, ss1)

    def idx_copy(g, u_v, sem):
        off = pl.multiple_of((h0 + g) * C, C)
        return pltpu.make_async_copy(u_hbm.at[pl.ds(off, C)], u_v, sem)

    def gather(u_v, rows_v, sem):
        return pltpu.make_async_copy(t_hbm.at[u_v], rows_v, sem)

    def store(g, to_v, sem):
        h = h0 + g
        j = h >> 4
        tc0 = (h & (CPP - 1)) * (C // 128)
        return pltpu.make_async_copy(
            to_v, out_hbm.at[j, :, pl.ds(tc0, C // 128), :, :], sem)

    def transpose(rows_v, to_v):
        iota = lax.iota(jnp.int32, LANES)

        @plsc.parallel_loop(0, D_MODEL, unroll=2)
        def _(s):
            dv = jnp.bitwise_and(iota + s, D_MODEL - 1)
            band_v = dv >> 3
            row_v = jnp.bitwise_and(dv, 7)
            for i0 in range(0, C // LANES, 4):
                vals = []
                for m in range(4):
                    tv = iota + (LANES * (i0 + m))
                    vals.append(plsc.load_gather(rows_v, [tv, dv]))
                for m in range(4):
                    i = i0 + m
                    tc_v = jnp.full((LANES,), (LANES * i) >> 7, jnp.int32)
                    col_v = iota + ((LANES * i) & 127)
                    plsc.store_scatter(
                        to_v, [band_v, tc_v, row_v, col_v], vals[m])

    for k in range(4):
        idx_copy(k, ub[k], sib[k]).start()
    for k in range(3):
        idx_copy(k, ub[k], sib[k]).wait()
        gather(ub[k], rb[k], sgb[k]).start()

    NQ = CH // 4

    @pl.loop(0, NQ)
    def _(q):
        a = q * 4
        for k in range(4):
            g = a + k
            gather(ub[k], rb[k], sgb[k]).wait()

            @pl.when(q < NQ - 1)
            def _():
                idx_copy(g + 4, ub[k], sib[k]).start()

            if k < 2:
                @pl.when(q > 0)
                def _():
                    store(g - 2, tb[k & 1], ssb[k & 1]).wait()
            else:
                store(g - 2, tb[k & 1], ssb[k & 1]).wait()

            transpose(rb[k], tb[k & 1])
            store(g, tb[k & 1], ssb[k & 1]).start()

            nk = (k + 3) & 3

            @pl.when(g + 3 < CH)
            def _():
                idx_copy(g + 3, ub[nk], sib[nk]).wait()
                gather(ub[nk], rb[nk], sgb[nk]).start()

    store(CH - 2, tb[0], ssb[0]).wait()
    store(CH - 1, tb[1], ssb[1]).wait()


def _sc_gather(u, t_lin):
    mesh = plsc.VectorSubcoreMesh(
        core_axis_name="c", subcore_axis_name="s",
        num_cores=NUM_CORES, num_subcores=NUM_SUBCORES)
    f = pl.kernel(
        _sc_body,
        out_type=jax.ShapeDtypeStruct(
            (J, D_MODEL // 8, B1 // 128, 8, 128), jnp.float32),
        mesh=mesh,
        scratch_types=(
            [pltpu.VMEM((C,), jnp.int32)] * 4
            + [pltpu.VMEM((C, D_MODEL), jnp.float32)] * 4
            + [pltpu.VMEM((D_MODEL // 8, C // 128, 8, 128), jnp.float32)] * 2
            + [pltpu.SemaphoreType.DMA] * 10
        ),
        compiler_params=pltpu.CompilerParams(
            use_tc_tiling_on_sc=False, needs_layout_passes=False,
            disable_bounds_checks=True),
    )
    return f(u, t_lin)


def kernel(x, table):
    xt = x.T.reshape(-1).astype(jnp.int32)
    q = jnp.bitwise_and(xt, WIN - 1)
    u = (xt - q) + 2 * jnp.bitwise_and(q, HW - 1) + (q >> int(math.log2(HW)))
    t2 = _build_t2(table.T)
    t_lin = t2.reshape(VP, D_MODEL)
    out5d = _sc_gather(u, t_lin)
    return jnp.transpose(out5d, (2, 4, 0, 1, 3)).reshape(B1, J, D_MODEL)


# speedup vs baseline: 2.2623x; 1.1602x over previous
import functools
import math

import jax
import jax.numpy as jnp
from jax import lax
from jax.experimental import pallas as pl
from jax.experimental.pallas import tpu as pltpu
from jax.experimental.pallas import tpu_sc as plsc

D_MODEL = 64
VOCAB = 1_000_000
SCALE = math.sqrt(D_MODEL)

WIN = 8192
HW = WIN // 2
NG = -(-VOCAB // WIN)
VP = NG * WIN

NUM_CORES = 2
NUM_SUBCORES = 16
LANES = 16
NW = NUM_CORES * NUM_SUBCORES

B1 = 4096
J = 200
STRIPE = B1 // NW

def _t2_body(a_ref, o_ref):
    o_ref[:, 0:D_MODEL] = jnp.swapaxes(a_ref[:, 0:HW], 0, 1) * SCALE
    o_ref[:, D_MODEL:2 * D_MODEL] = jnp.swapaxes(a_ref[:, HW:WIN], 0, 1) * SCALE


def _build_t2(tt):
    return pl.pallas_call(
        _t2_body,
        grid=(NG,),
        in_specs=[pl.BlockSpec((D_MODEL, WIN), lambda g: (0, g))],
        out_specs=pl.BlockSpec((HW, 2 * D_MODEL), lambda g: (g, 0)),
        out_shape=jax.ShapeDtypeStruct((VP // 2, 2 * D_MODEL), jnp.float32),
    )(tt)


C = 256
CPP = B1 // C
CH = (J * CPP) // NW
NPAIR = CH // 2


def _sc_body(u_hbm, t_hbm, out_hbm,
             u0, u1, u2, u3, rows0, rows1, rows2, rows3, to0, to1,
             si0, si1, si2, si3, sg0, sg1, sg2, sg3, ss0, ss1):
    wid = lax.axis_index("s") * NUM_CORES + lax.axis_index("c")
    h0 = wid * CH
    ub = (u0, u1, u2, u3)
    rb = (rows0, rows1, rows2, rows3)
    tb = (to0, to1)
    sib = (si0, si1, si2, si3)
    sgb = (sg0, sg1, sg2, sg3)
    ssb = (ss0, ss1)

    def idx_copy(g, u_v, sem):
        off = pl.multiple_of((h0 + g) * C, C)
        return pltpu.make_async_copy(u_hbm.at[pl.ds(off, C)], u_v, sem)

    def gather(u_v, rows_v, sem):
        return pltpu.make_async_copy(t_hbm.at[u_v], rows_v, sem)

    def store(g, to_v, sem):
        h = h0 + g
        j = h >> 4
        tc0 = (h & (CPP - 1)) * (C // 128)
        return pltpu.make_async_copy(
            to_v, out_hbm.at[j, :, pl.ds(tc0, C // 128), :, :], sem)

    def transpose(rows_v, to_v):
        iota = lax.iota(jnp.int32, LANES)

        @plsc.parallel_loop(0, D_MODEL, unroll=2)
        def _(s):
            dv = jnp.bitwise_and(iota + s, D_MODEL - 1)
            band_v = dv >> 3
            row_v = jnp.bitwise_and(dv, 7)
            for i0 in range(0, C // LANES, 4):
                vals = []
                for m in range(4):
                    tv = iota + (LANES * (i0 + m))
                    vals.append(plsc.load_gather(rows_v, [tv, dv]))
                for m in range(4):
                    i = i0 + m
                    tc_v = jnp.full((LANES,), (LANES * i) >> 7, jnp.int32)
                    col_v = iota + ((LANES * i) & 127)
                    plsc.store_scatter(
                        to_v, [band_v, tc_v, row_v, col_v], vals[m])

    for k in range(4):
        idx_copy(k, ub[k], sib[k]).start()
    for k in range(3):
        idx_copy(k, ub[k], sib[k]).wait()
        gather(ub[k], rb[k], sgb[k]).start()

    NQ = CH // 4

    @pl.loop(0, NQ)
    def _(q):
        a = q * 4
        for k in range(4):
            g = a + k
            gather(ub[k], rb[k], sgb[k]).wait()

            @pl.when(q < NQ - 1)
            def _():
                idx_copy(g + 4, ub[k], sib[k]).start()

            if k < 2:
                @pl.when(q > 0)
                def _():
                    store(g - 2, tb[k & 1], ssb[k & 1]).wait()
            else:
                store(g - 2, tb[k & 1], ssb[k & 1]).wait()

            transpose(rb[k], tb[k & 1])
            store(g, tb[k & 1], ssb[k & 1]).start()

            nk = (k + 3) & 3

            @pl.when(g + 3 < CH)
            def _():
                idx_copy(g + 3, ub[nk], sib[nk]).wait()
                gather(ub[nk], rb[nk], sgb[nk]).start()

    store(CH - 2, tb[0], ssb[0]).wait()
    store(CH - 1, tb[1], ssb[1]).wait()


def _sc_gather(u, t_lin):
    mesh = plsc.VectorSubcoreMesh(
        core_axis_name="c", subcore_axis_name="s",
        num_cores=NUM_CORES, num_subcores=NUM_SUBCORES)
    f = pl.kernel(
        _sc_body,
        out_type=jax.ShapeDtypeStruct(
            (J, D_MODEL // 8, B1 // 128, 8, 128), jnp.float32),
        mesh=mesh,
        scratch_types=(
            [pltpu.VMEM((C,), jnp.int32)] * 4
            + [pltpu.VMEM((C, D_MODEL), jnp.float32)] * 4
            + [pltpu.VMEM((D_MODEL // 8, C // 128, 8, 128), jnp.float32)] * 2
            + [pltpu.SemaphoreType.DMA] * 10
        ),
        compiler_params=pltpu.CompilerParams(
            use_tc_tiling_on_sc=False, needs_layout_passes=False,
            disable_bounds_checks=True),
    )
    return f(u, t_lin)


def kernel(x, table):
    xt = x.T.reshape(-1).astype(jnp.int32)
    q = jnp.bitwise_and(xt, WIN - 1)
    u = (xt - q) + 2 * jnp.bitwise_and(q, HW - 1) + (q >> int(math.log2(HW)))
    t2 = _build_t2(table.T)
    t_lin = t2.reshape(VP, D_MODEL)
    out5d = _sc_gather(u, t_lin)
    return jnp.transpose(out5d, (2, 4, 0, 1, 3)).reshape(B1, J, D_MODEL)


# speedup vs baseline: 2.4537x; 1.0846x over previous
import functools
import math

import jax
import jax.numpy as jnp
from jax import lax
from jax.experimental import pallas as pl
from jax.experimental.pallas import tpu as pltpu
from jax.experimental.pallas import tpu_sc as plsc

D_MODEL = 64
VOCAB = 1_000_000
SCALE = math.sqrt(D_MODEL)

WIN = 16384
HW = WIN // 2
NG = -(-VOCAB // WIN)
VP = NG * WIN

NUM_CORES = 2
NUM_SUBCORES = 16
LANES = 16
NW = NUM_CORES * NUM_SUBCORES

B1 = 4096
J = 200
STRIPE = B1 // NW

def _t2_body(a_ref, o_ref):
    o_ref[:, 0:D_MODEL] = jnp.swapaxes(a_ref[:, 0:HW], 0, 1) * SCALE
    o_ref[:, D_MODEL:2 * D_MODEL] = jnp.swapaxes(a_ref[:, HW:WIN], 0, 1) * SCALE


def _build_t2(tt):
    return pl.pallas_call(
        _t2_body,
        grid=(NG,),
        in_specs=[pl.BlockSpec((D_MODEL, WIN), lambda g: (0, g))],
        out_specs=pl.BlockSpec((HW, 2 * D_MODEL), lambda g: (g, 0)),
        out_shape=jax.ShapeDtypeStruct((VP // 2, 2 * D_MODEL), jnp.float32),
    )(tt)


C = 256
CPP = B1 // C
CH = (J * CPP) // NW
NPAIR = CH // 2


def _sc_body(u_hbm, t_hbm, out_hbm,
             u0, u1, u2, u3, rows0, rows1, rows2, rows3, to0, to1,
             si0, si1, si2, si3, sg0, sg1, sg2, sg3, ss0, ss1):
    wid = lax.axis_index("s") * NUM_CORES + lax.axis_index("c")
    h0 = wid * CH
    ub = (u0, u1, u2, u3)
    rb = (rows0, rows1, rows2, rows3)
    tb = (to0, to1)
    sib = (si0, si1, si2, si3)
    sgb = (sg0, sg1, sg2, sg3)
    ssb = (ss0, ss1)

    def idx_copy(g, u_v, sem):
        off = pl.multiple_of((h0 + g) * C, C)
        return pltpu.make_async_copy(u_hbm.at[pl.ds(off, C)], u_v, sem)

    def gather(u_v, rows_v, sem):
        return pltpu.make_async_copy(t_hbm.at[u_v], rows_v, sem)

    def store(g, to_v, sem):
        h = h0 + g
        j = h >> 4
        tc0 = (h & (CPP - 1)) * (C // 128)
        return pltpu.make_async_copy(
            to_v, out_hbm.at[j, :, pl.ds(tc0, C // 128), :, :], sem)

    def transpose(rows_v, to_v):
        iota = lax.iota(jnp.int32, LANES)

        @plsc.parallel_loop(0, D_MODEL, unroll=2)
        def _(s):
            dv = jnp.bitwise_and(iota + s, D_MODEL - 1)
            band_v = dv >> 3
            row_v = jnp.bitwise_and(dv, 7)
            for i0 in range(0, C // LANES, 4):
                vals = []
                for m in range(4):
                    tv = iota + (LANES * (i0 + m))
                    vals.append(plsc.load_gather(rows_v, [tv, dv]))
                for m in range(4):
                    i = i0 + m
                    tc_v = jnp.full((LANES,), (LANES * i) >> 7, jnp.int32)
                    col_v = iota + ((LANES * i) & 127)
                    plsc.store_scatter(
                        to_v, [band_v, tc_v, row_v, col_v], vals[m])

    for k in range(4):
        idx_copy(k, ub[k], sib[k]).start()
    for k in range(3):
        idx_copy(k, ub[k], sib[k]).wait()
        gather(ub[k], rb[k], sgb[k]).start()

    NQ = CH // 4

    @pl.loop(0, NQ)
    def _(q):
        a = q * 4
        for k in range(4):
            g = a + k
            gather(ub[k], rb[k], sgb[k]).wait()

            @pl.when(q < NQ - 1)
            def _():
                idx_copy(g + 4, ub[k], sib[k]).start()

            if k < 2:
                @pl.when(q > 0)
                def _():
                    store(g - 2, tb[k & 1], ssb[k & 1]).wait()
            else:
                store(g - 2, tb[k & 1], ssb[k & 1]).wait()

            transpose(rb[k], tb[k & 1])
            store(g, tb[k & 1], ssb[k & 1]).start()

            nk = (k + 3) & 3

            @pl.when(g + 3 < CH)
            def _():
                idx_copy(g + 3, ub[nk], sib[nk]).wait()
                gather(ub[nk], rb[nk], sgb[nk]).start()

    store(CH - 2, tb[0], ssb[0]).wait()
    store(CH - 1, tb[1], ssb[1]).wait()


def _sc_gather(u, t_lin):
    mesh = plsc.VectorSubcoreMesh(
        core_axis_name="c", subcore_axis_name="s",
        num_cores=NUM_CORES, num_subcores=NUM_SUBCORES)
    f = pl.kernel(
        _sc_body,
        out_type=jax.ShapeDtypeStruct(
            (J, D_MODEL // 8, B1 // 128, 8, 128), jnp.float32),
        mesh=mesh,
        scratch_types=(
            [pltpu.VMEM((C,), jnp.int32)] * 4
            + [pltpu.VMEM((C, D_MODEL), jnp.float32)] * 4
            + [pltpu.VMEM((D_MODEL // 8, C // 128, 8, 128), jnp.float32)] * 2
            + [pltpu.SemaphoreType.DMA] * 10
        ),
        compiler_params=pltpu.CompilerParams(
            use_tc_tiling_on_sc=False, needs_layout_passes=False,
            disable_bounds_checks=True),
    )
    return f(u, t_lin)


def kernel(x, table):
    xt = x.T.reshape(-1).astype(jnp.int32)
    q = jnp.bitwise_and(xt, WIN - 1)
    u = (xt - q) + 2 * jnp.bitwise_and(q, HW - 1) + (q >> int(math.log2(HW)))
    t2 = _build_t2(table.T)
    t_lin = t2.reshape(VP, D_MODEL)
    out5d = _sc_gather(u, t_lin)
    return jnp.transpose(out5d, (2, 4, 0, 1, 3)).reshape(B1, J, D_MODEL)


# speedup vs baseline: 2.5429x; 1.0364x over previous
import functools
import math

import jax
import jax.numpy as jnp
from jax import lax
from jax.experimental import pallas as pl
from jax.experimental.pallas import tpu as pltpu
from jax.experimental.pallas import tpu_sc as plsc

D_MODEL = 64
VOCAB = 1_000_000
SCALE = math.sqrt(D_MODEL)

WIN = 32768
HW = WIN // 2
NG = -(-VOCAB // WIN)
VP = NG * WIN

NUM_CORES = 2
NUM_SUBCORES = 16
LANES = 16
NW = NUM_CORES * NUM_SUBCORES

B1 = 4096
J = 200
STRIPE = B1 // NW

def _t2_body(a_ref, o_ref):
    o_ref[:, 0:D_MODEL] = jnp.swapaxes(a_ref[:, 0:HW], 0, 1) * SCALE
    o_ref[:, D_MODEL:2 * D_MODEL] = jnp.swapaxes(a_ref[:, HW:WIN], 0, 1) * SCALE


def _build_t2(tt):
    return pl.pallas_call(
        _t2_body,
        grid=(NG,),
        in_specs=[pl.BlockSpec((D_MODEL, WIN), lambda g: (0, g))],
        out_specs=pl.BlockSpec((HW, 2 * D_MODEL), lambda g: (g, 0)),
        out_shape=jax.ShapeDtypeStruct((VP // 2, 2 * D_MODEL), jnp.float32),
    )(tt)


C = 256
CPP = B1 // C
CH = (J * CPP) // NW
NPAIR = CH // 2


def _sc_body(u_hbm, t_hbm, out_hbm,
             u0, u1, u2, u3, rows0, rows1, rows2, rows3, to0, to1,
             si0, si1, si2, si3, sg0, sg1, sg2, sg3, ss0, ss1):
    wid = lax.axis_index("s") * NUM_CORES + lax.axis_index("c")
    h0 = wid * CH
    ub = (u0, u1, u2, u3)
    rb = (rows0, rows1, rows2, rows3)
    tb = (to0, to1)
    sib = (si0, si1, si2, si3)
    sgb = (sg0, sg1, sg2, sg3)
    ssb = (ss0, ss1)

    def idx_copy(g, u_v, sem):
        off = pl.multiple_of((h0 + g) * C, C)
        return pltpu.make_async_copy(u_hbm.at[pl.ds(off, C)], u_v, sem)

    def gather(u_v, rows_v, sem):
        return pltpu.make_async_copy(t_hbm.at[u_v], rows_v, sem)

    def store(g, to_v, sem):
        h = h0 + g
        j = h >> 4
        tc0 = (h & (CPP - 1)) * (C // 128)
        return pltpu.make_async_copy(
            to_v, out_hbm.at[j, :, pl.ds(tc0, C // 128), :, :], sem)

    def transpose(rows_v, to_v):
        iota = lax.iota(jnp.int32, LANES)

        @plsc.parallel_loop(0, D_MODEL, unroll=2)
        def _(s):
            dv = jnp.bitwise_and(iota + s, D_MODEL - 1)
            band_v = dv >> 3
            row_v = jnp.bitwise_and(dv, 7)
            for i0 in range(0, C // LANES, 4):
                vals = []
                for m in range(4):
                    tv = iota + (LANES * (i0 + m))
                    vals.append(plsc.load_gather(rows_v, [tv, dv]))
                for m in range(4):
                    i = i0 + m
                    tc_v = jnp.full((LANES,), (LANES * i) >> 7, jnp.int32)
                    col_v = iota + ((LANES * i) & 127)
                    plsc.store_scatter(
                        to_v, [band_v, tc_v, row_v, col_v], vals[m])

    for k in range(4):
        idx_copy(k, ub[k], sib[k]).start()
    for k in range(3):
        idx_copy(k, ub[k], sib[k]).wait()
        gather(ub[k], rb[k], sgb[k]).start()

    NQ = CH // 4

    @pl.loop(0, NQ)
    def _(q):
        a = q * 4
        for k in range(4):
            g = a + k
            gather(ub[k], rb[k], sgb[k]).wait()

            @pl.when(q < NQ - 1)
            def _():
                idx_copy(g + 4, ub[k], sib[k]).start()

            if k < 2:
                @pl.when(q > 0)
                def _():
                    store(g - 2, tb[k & 1], ssb[k & 1]).wait()
            else:
                store(g - 2, tb[k & 1], ssb[k & 1]).wait()

            transpose(rb[k], tb[k & 1])
            store(g, tb[k & 1], ssb[k & 1]).start()

            nk = (k + 3) & 3

            @pl.when(g + 3 < CH)
            def _():
                idx_copy(g + 3, ub[nk], sib[nk]).wait()
                gather(ub[nk], rb[nk], sgb[nk]).start()

    store(CH - 2, tb[0], ssb[0]).wait()
    store(CH - 1, tb[1], ssb[1]).wait()


def _sc_gather(u, t_lin):
    mesh = plsc.VectorSubcoreMesh(
        core_axis_name="c", subcore_axis_name="s",
        num_cores=NUM_CORES, num_subcores=NUM_SUBCORES)
    f = pl.kernel(
        _sc_body,
        out_type=jax.ShapeDtypeStruct(
            (J, D_MODEL // 8, B1 // 128, 8, 128), jnp.float32),
        mesh=mesh,
        scratch_types=(
            [pltpu.VMEM((C,), jnp.int32)] * 4
            + [pltpu.VMEM((C, D_MODEL), jnp.float32)] * 4
            + [pltpu.VMEM((D_MODEL // 8, C // 128, 8, 128), jnp.float32)] * 2
            + [pltpu.SemaphoreType.DMA] * 10
        ),
        compiler_params=pltpu.CompilerParams(
            use_tc_tiling_on_sc=False, needs_layout_passes=False,
            disable_bounds_checks=True),
    )
    return f(u, t_lin)


def kernel(x, table):
    xt = x.T.reshape(-1).astype(jnp.int32)
    q = jnp.bitwise_and(xt, WIN - 1)
    u = (xt - q) + 2 * jnp.bitwise_and(q, HW - 1) + (q >> int(math.log2(HW)))
    t2 = _build_t2(table.T)
    t_lin = t2.reshape(VP, D_MODEL)
    out5d = _sc_gather(u, t_lin)
    return jnp.transpose(out5d, (2, 4, 0, 1, 3)).reshape(B1, J, D_MODEL)
